# baseline (speedup 1.0000x reference)
import functools

import jax
import jax.numpy as jnp
from jax import lax
from jax.experimental import pallas as pl
from jax.experimental.pallas import tpu as pltpu
from jax.experimental.pallas import tpu_sc as plsc

NUM_CORES = 2
NUM_SUBCORES = 16
NUM_WORKERS = NUM_CORES * NUM_SUBCORES
LANES = 16
BATCH_N = 16384
FEAT = 64
ROWS_PER_W = BATCH_N // NUM_WORKERS


def _gather_body(user_hbm, item_hbm, uemb_hbm, iemb_hbm,
                 urows_hbm, irows_hbm,
                 uidx_v, iidx_v, sem):
    wid = lax.axis_index("s") * NUM_CORES + lax.axis_index("c")
    base = wid * ROWS_PER_W

    pltpu.sync_copy(user_hbm.at[pl.ds(base, ROWS_PER_W)], uidx_v)
    pltpu.sync_copy(item_hbm.at[pl.ds(base, ROWS_PER_W)], iidx_v)

    def fire(g, _):
        uvec = uidx_v[pl.ds(g * LANES, LANES)]
        ivec = iidx_v[pl.ds(g * LANES, LANES)]
        for rr in range(LANES):
            r = base + g * LANES + rr
            pltpu.async_copy(uemb_hbm.at[uvec[rr]], urows_hbm.at[r], sem)
            pltpu.async_copy(iemb_hbm.at[ivec[rr]], irows_hbm.at[r], sem)
        return ()

    lax.fori_loop(0, ROWS_PER_W // LANES, fire, ())

    def drain(r, _):
        pltpu.make_async_copy(
            uemb_hbm.at[0], urows_hbm.at[base + r], sem).wait()
        pltpu.make_async_copy(
            iemb_hbm.at[0], irows_hbm.at[base + r], sem).wait()
        return ()

    lax.fori_loop(0, ROWS_PER_W, drain, ())


def _dot_body(u_ref, i_ref, o_ref):
    o_ref[...] = jnp.sum(u_ref[...] * i_ref[...], axis=1)


@jax.jit
def kernel(user, item, users_emb, items_emb):
    mesh = plsc.VectorSubcoreMesh(core_axis_name="c", subcore_axis_name="s")
    gather = pl.kernel(
        _gather_body,
        out_type=(jax.ShapeDtypeStruct((BATCH_N, FEAT), jnp.float32),
                  jax.ShapeDtypeStruct((BATCH_N, FEAT), jnp.float32)),
        mesh=mesh,
        scratch_types=[
            pltpu.VMEM((ROWS_PER_W,), jnp.int32),
            pltpu.VMEM((ROWS_PER_W,), jnp.int32),
            pltpu.SemaphoreType.DMA,
        ],
        compiler_params=pltpu.CompilerParams(needs_layout_passes=False),
    )
    urows, irows = gather(user.astype(jnp.int32), item.astype(jnp.int32),
                          users_emb, items_emb)

    nblk = 8
    dot = pl.pallas_call(
        _dot_body,
        out_shape=jax.ShapeDtypeStruct((BATCH_N,), jnp.float32),
        grid=(nblk,),
        in_specs=[
            pl.BlockSpec((BATCH_N // nblk, FEAT), lambda b: (b, 0)),
            pl.BlockSpec((BATCH_N // nblk, FEAT), lambda b: (b, 0)),
        ],
        out_specs=pl.BlockSpec((BATCH_N // nblk,), lambda b: (b,)),
    )
    return dot(urows, irows)


# speedup vs baseline: 1.7180x; 1.7180x over previous
import functools

import jax
import jax.numpy as jnp
from jax import lax
from jax.experimental import pallas as pl
from jax.experimental.pallas import tpu as pltpu
from jax.experimental.pallas import tpu_sc as plsc

NUM_CORES = 2
NUM_SUBCORES = 16
NUM_WORKERS = NUM_CORES * NUM_SUBCORES
LANES = 16
BATCH_N = 16384
FEAT = 64
ROWS_PER_W = BATCH_N // NUM_WORKERS
CHUNK = 256
NCHUNK = ROWS_PER_W // CHUNK


def _body(user_hbm, item_hbm, uemb_hbm, iemb_hbm, out_hbm,
          uidx_v, iidx_v, urows_v, irows_v, out_v, sem):
    wid = lax.axis_index("s") * NUM_CORES + lax.axis_index("c")
    base = wid * ROWS_PER_W

    pltpu.sync_copy(user_hbm.at[pl.ds(base, ROWS_PER_W)], uidx_v)
    pltpu.sync_copy(item_hbm.at[pl.ds(base, ROWS_PER_W)], iidx_v)

    lane = lax.iota(jnp.int32, LANES)

    def chunk_body(c, _):
        lo = c * CHUNK

        def fire(g, _):
            uvec = uidx_v[pl.ds(lo + g * LANES, LANES)]
            ivec = iidx_v[pl.ds(lo + g * LANES, LANES)]
            for rr in range(LANES):
                k = g * LANES + rr
                pltpu.async_copy(uemb_hbm.at[uvec[rr]],
                                 urows_v.at[k, 0], sem)
                pltpu.async_copy(iemb_hbm.at[ivec[rr]],
                                 irows_v.at[k, 0], sem)
            return ()

        lax.fori_loop(0, CHUNK // LANES, fire, ())

        def drain(k, _):
            pltpu.make_async_copy(
                uemb_hbm.at[0], urows_v.at[k, 0], sem).wait()
            pltpu.make_async_copy(
                iemb_hbm.at[0], irows_v.at[k, 0], sem).wait()
            return ()

        lax.fori_loop(0, CHUNK, drain, ())

        def blk_body(blk, _):
            acc16 = jnp.zeros((LANES,), jnp.float32)
            for rr in range(LANES):
                k = blk * LANES + rr
                parts = []
                for j in range(FEAT // LANES):
                    u = urows_v[k, 0, pl.ds(j * LANES, LANES)]
                    i = irows_v[k, 0, pl.ds(j * LANES, LANES)]
                    parts.append(u * i)
                s = (parts[0] + parts[1]) + (parts[2] + parts[3])
                tot = jnp.sum(s)
                acc16 = jnp.where(lane == rr, tot, acc16)
            out_v[pl.ds(lo + blk * LANES, LANES)] = acc16
            return ()

        lax.fori_loop(0, CHUNK // LANES, blk_body, ())
        return ()

    lax.fori_loop(0, NCHUNK, chunk_body, ())

    pltpu.sync_copy(out_v, out_hbm.at[pl.ds(base, ROWS_PER_W)])


@jax.jit
def kernel(user, item, users_emb, items_emb):
    mesh = plsc.VectorSubcoreMesh(core_axis_name="c", subcore_axis_name="s")
    k = pl.kernel(
        _body,
        out_type=jax.ShapeDtypeStruct((BATCH_N,), jnp.float32),
        mesh=mesh,
        scratch_types=[
            pltpu.VMEM((ROWS_PER_W,), jnp.int32),
            pltpu.VMEM((ROWS_PER_W,), jnp.int32),
            pltpu.VMEM((CHUNK, 1, FEAT), jnp.float32),
            pltpu.VMEM((CHUNK, 1, FEAT), jnp.float32),
            pltpu.VMEM((ROWS_PER_W,), jnp.float32),
            pltpu.SemaphoreType.DMA,
        ],
        compiler_params=pltpu.CompilerParams(needs_layout_passes=False),
    )
    return k(user.astype(jnp.int32), item.astype(jnp.int32),
             users_emb, items_emb)
